# baseline (speedup 1.0000x reference)
import functools

import jax
import jax.numpy as jnp
from jax import lax
from jax.experimental import pallas as pl
from jax.experimental.pallas import tpu as pltpu
from jax.experimental.pallas import tpu_sc as plsc

_NC = 2
_NS = 16
_L = 16
_NW = _NC * _NS
_B = 4096
_F = 64
_BPW = _B // _NW
_G = _BPW // _L


def _body(user_hbm, pos_hbm, neg_hbm, uw_hbm, iw_hbm, out_hbm,
          uidx, pidx, nidx, urows, prows, nrows, tbuf, scores,
          s0, s1, s2):
    wid = lax.axis_index("s") * _NC + lax.axis_index("c")
    base = wid * _BPW

    pltpu.sync_copy(user_hbm.at[pl.ds(base, _BPW)], uidx)
    pltpu.sync_copy(pos_hbm.at[pl.ds(base, _BPW)], pidx)
    pltpu.sync_copy(neg_hbm.at[pl.ds(base, _BPW)], nidx)

    cu = pltpu.async_copy(uw_hbm.at[uidx], urows, s0)
    cp = pltpu.async_copy(iw_hbm.at[pidx], prows, s1)
    cn = pltpu.async_copy(iw_hbm.at[nidx], nrows, s2)
    cu.wait()
    cp.wait()
    cn.wait()

    lanes = lax.iota(jnp.int32, _L)

    def group(g, carry):
        for j in range(_L):
            r = g * _L + j
            acc = jnp.zeros((_L,), jnp.float32)
            for f in range(0, _F, _L):
                u = urows[r, pl.ds(f, _L)]
                p = prows[r, pl.ds(f, _L)]
                n = nrows[r, pl.ds(f, _L)]
                acc = acc + u * (p - n)
            tbuf[pl.ds(j * _L, _L)] = acc
        tot = jnp.zeros((_L,), jnp.float32)
        for c in range(_L):
            tot = tot + plsc.load_gather(tbuf, [lanes * _L + c])
        res = 2.0 - 1.0 / (1.0 + jnp.exp(-tot))
        scores[pl.ds(g * _L, _L)] = res
        return carry

    lax.fori_loop(0, _G, group, 0)
    pltpu.sync_copy(scores, out_hbm.at[pl.ds(base, _BPW)])


@functools.lru_cache(maxsize=1)
def _build():
    return pl.kernel(
        _body,
        out_type=jax.ShapeDtypeStruct((_B,), jnp.float32),
        mesh=plsc.VectorSubcoreMesh(
            core_axis_name="c", subcore_axis_name="s",
            num_cores=_NC, num_subcores=_NS),
        scratch_types=[
            pltpu.VMEM((_BPW,), jnp.int32),
            pltpu.VMEM((_BPW,), jnp.int32),
            pltpu.VMEM((_BPW,), jnp.int32),
            pltpu.VMEM((_BPW, _F), jnp.float32),
            pltpu.VMEM((_BPW, _F), jnp.float32),
            pltpu.VMEM((_BPW, _F), jnp.float32),
            pltpu.VMEM((_L * _L,), jnp.float32),
            pltpu.VMEM((_BPW,), jnp.float32),
            pltpu.SemaphoreType.DMA,
            pltpu.SemaphoreType.DMA,
            pltpu.SemaphoreType.DMA,
        ],
        compiler_params=pltpu.CompilerParams(
            needs_layout_passes=False, use_tc_tiling_on_sc=False),
    )


@jax.jit
def kernel(user, posItem, negItem, user_W, item_W):
    out = _build()(user, posItem, negItem, user_W, item_W)
    return out.reshape(-1, 1)


# speedup vs baseline: 1.4247x; 1.4247x over previous
import functools

import jax
import jax.numpy as jnp
from jax import lax
from jax.experimental import pallas as pl
from jax.experimental.pallas import tpu as pltpu
from jax.experimental.pallas import tpu_sc as plsc

_NC = 2
_NS = 16
_L = 16
_NW = _NC * _NS
_B = 4096
_F = 64
_BPW = _B // _NW
_G = _BPW // _L


def _body(user_hbm, pos_hbm, neg_hbm, uw_hbm, iw_hbm, out_hbm,
          vidx, urows, prows, nrows, tbuf, scores,
          su, sp, sn):
    wid = lax.axis_index("s") * _NC + lax.axis_index("c")
    base = wid * _BPW

    pltpu.sync_copy(user_hbm.at[pl.ds(base, _BPW)], vidx.at[0])
    pltpu.sync_copy(pos_hbm.at[pl.ds(base, _BPW)], vidx.at[1])
    pltpu.sync_copy(neg_hbm.at[pl.ds(base, _BPW)], vidx.at[2])

    def fire(c, carry):
        vu = vidx[0, pl.ds(c * _L, _L)]
        vp = vidx[1, pl.ds(c * _L, _L)]
        vn = vidx[2, pl.ds(c * _L, _L)]
        for j in range(_L):
            r = c * _L + j
            pltpu.async_copy(uw_hbm.at[vu[j]], urows.at[r], su)
            pltpu.async_copy(iw_hbm.at[vp[j]], prows.at[r], sp)
            pltpu.async_copy(iw_hbm.at[vn[j]], nrows.at[r], sn)
        return carry

    lax.fori_loop(0, _G, fire, 0)
    pltpu.make_async_copy(uw_hbm.at[pl.ds(0, _BPW)], urows, su).wait()
    pltpu.make_async_copy(iw_hbm.at[pl.ds(0, _BPW)], prows, sp).wait()
    pltpu.make_async_copy(iw_hbm.at[pl.ds(0, _BPW)], nrows, sn).wait()

    lanes = lax.iota(jnp.int32, _L)

    def group(g, carry):
        for j in range(_L):
            r = g * _L + j
            acc = jnp.zeros((_L,), jnp.float32)
            for f in range(0, _F, _L):
                u = urows[r, pl.ds(f, _L)]
                p = prows[r, pl.ds(f, _L)]
                n = nrows[r, pl.ds(f, _L)]
                acc = acc + u * (p - n)
            tbuf[pl.ds(j * _L, _L)] = acc
        tot = jnp.zeros((_L,), jnp.float32)
        for c in range(_L):
            tot = tot + plsc.load_gather(tbuf, [lanes * _L + c])
        res = 2.0 - 1.0 / (1.0 + jnp.exp(-tot))
        scores[pl.ds(g * _L, _L)] = res
        return carry

    lax.fori_loop(0, _G, group, 0)
    pltpu.sync_copy(scores, out_hbm.at[pl.ds(base, _BPW)])


@functools.lru_cache(maxsize=1)
def _build():
    return pl.kernel(
        _body,
        out_type=jax.ShapeDtypeStruct((_B,), jnp.float32),
        mesh=plsc.VectorSubcoreMesh(
            core_axis_name="c", subcore_axis_name="s",
            num_cores=_NC, num_subcores=_NS),
        scratch_types=[
            pltpu.VMEM((3, _BPW), jnp.int32),
            pltpu.VMEM((_BPW, _F), jnp.float32),
            pltpu.VMEM((_BPW, _F), jnp.float32),
            pltpu.VMEM((_BPW, _F), jnp.float32),
            pltpu.VMEM((_L * _L,), jnp.float32),
            pltpu.VMEM((_BPW,), jnp.float32),
            pltpu.SemaphoreType.DMA,
            pltpu.SemaphoreType.DMA,
            pltpu.SemaphoreType.DMA,
        ],
        compiler_params=pltpu.CompilerParams(needs_layout_passes=False),
    )


@jax.jit
def kernel(user, posItem, negItem, user_W, item_W):
    out = _build()(user, posItem, negItem, user_W, item_W)
    return out.reshape(-1, 1)
